# baseline (speedup 1.0000x reference)
import functools

import jax
import jax.numpy as jnp
from jax import lax
from jax.experimental import pallas as pl
from jax.experimental.pallas import tpu as pltpu
from jax.experimental.pallas import tpu_sc as plsc

_N = 524288
_K = 8
_C = 256
_B = 8192
_NB = _N // _B

_NC = 2
_NS = 16
_NW = _NC * _NS
_ROWS_PER_W = _N // _NW
_CHUNK = 4096


def _score_argmax_body(xt_ref, g2_ref, norm_ref, idx_ref):
    sc = jnp.dot(g2_ref[...], xt_ref[...], preferred_element_type=jnp.float32)
    sc = sc - norm_ref[...]
    m = jnp.max(sc, axis=0, keepdims=True)
    ii = lax.broadcasted_iota(jnp.int32, (_C, _B), 0)
    cand = jnp.where(sc == m, ii, _C)
    idx_ref[0] = jnp.min(cand, axis=0).reshape(_B // 128, 128)


def _tc_score_argmax(xt, g2, norm2):
    return pl.pallas_call(
        _score_argmax_body,
        grid=(_NB,),
        in_specs=[
            pl.BlockSpec((_K, _B), lambda i: (0, i)),
            pl.BlockSpec((_C, _K), lambda i: (0, 0)),
            pl.BlockSpec((_C, 1), lambda i: (0, 0)),
        ],
        out_specs=pl.BlockSpec((1, _B // 128, 128), lambda i: (i, 0, 0)),
        out_shape=jax.ShapeDtypeStruct((_NB, _B // 128, 128), jnp.int32),
    )(xt, g2, norm2)


def _sc_gather(idx_flat, table):
    mesh = plsc.VectorSubcoreMesh(core_axis_name="c", subcore_axis_name="s")

    @functools.partial(
        pl.kernel,
        mesh=mesh,
        out_type=jax.ShapeDtypeStruct((_N * _K,), jnp.float32),
        scratch_types=[
            pltpu.VMEM((_ROWS_PER_W,), jnp.int32),
            pltpu.VMEM((_C * _K,), jnp.float32),
            pltpu.VMEM((_CHUNK * _K,), jnp.float32),
        ],
        compiler_params=pltpu.CompilerParams(needs_layout_passes=False),
    )
    def k(idx_hbm, table_hbm, out_hbm, idx_v, table_v, rows_v):
        wid = lax.axis_index("s") * _NC + lax.axis_index("c")
        row_base = wid * _ROWS_PER_W
        pltpu.sync_copy(table_hbm, table_v)
        pltpu.sync_copy(
            idx_hbm.at[pl.ds(pl.multiple_of(row_base, 128), _ROWS_PER_W)],
            idx_v)

        pos0 = lax.iota(jnp.int32, 16) * _K

        def chunk(c, carry):
            def body16(c2, carry2):
                coff = pl.multiple_of(c * _CHUNK + c2 * 16, 8)
                rvec = idx_v[pl.ds(coff, 16)] * _K
                pos = pos0 + c2 * (16 * _K)
                for kk in range(_K):
                    vals = plsc.load_gather(table_v, [rvec + kk])
                    plsc.store_scatter(rows_v, [pos + kk], vals)
                return carry2

            lax.fori_loop(0, _CHUNK // 16, body16, 0, unroll=2)
            ooff = pl.multiple_of((row_base + c * _CHUNK) * _K, 128)
            pltpu.sync_copy(
                rows_v.at[pl.ds(0, _CHUNK * _K)],
                out_hbm.at[pl.ds(ooff, _CHUNK * _K)])
            return carry

        lax.fori_loop(0, _ROWS_PER_W // _CHUNK, chunk, 0)

    return k(idx_flat, table)


def kernel(X, grid, grid_norm):
    xt = X.T
    g2 = 2.0 * grid
    norm2 = grid_norm.reshape(_C, 1)
    idx3 = _tc_score_argmax(xt, g2, norm2)
    idx_flat = idx3.reshape(_N)
    quantized = _sc_gather(idx_flat, grid.reshape(_C * _K)).reshape(_N, _K)
    return (quantized, idx_flat.astype(jnp.uint8))


# speedup vs baseline: 2.1378x; 2.1378x over previous
import functools

import jax
import jax.numpy as jnp
from jax import lax
from jax.experimental import pallas as pl
from jax.experimental.pallas import tpu as pltpu
from jax.experimental.pallas import tpu_sc as plsc

_N = 524288
_K = 8
_C = 256
_B = 8192
_NB = _N // _B

_NC = 2
_NS = 16
_NW = _NC * _NS
_ROWS_PER_W = _N // _NW
_CHUNK = 4096


def _score_argmax_body(xt_ref, g2_ref, norm_ref, idx_ref):
    sc = jnp.dot(g2_ref[...], xt_ref[...], preferred_element_type=jnp.float32)
    sc = sc - norm_ref[...]
    m = jnp.max(sc, axis=0, keepdims=True)
    ii = lax.broadcasted_iota(jnp.int32, (_C, _B), 0)
    cand = jnp.where(sc == m, ii, _C)
    idx_ref[0] = jnp.min(cand, axis=0).reshape(_B // 128, 128)


def _tc_score_argmax(xt, g2, norm2):
    return pl.pallas_call(
        _score_argmax_body,
        grid=(_NB,),
        in_specs=[
            pl.BlockSpec((_K, _B), lambda i: (0, i)),
            pl.BlockSpec((_C, _K), lambda i: (0, 0)),
            pl.BlockSpec((_C, 1), lambda i: (0, 0)),
        ],
        out_specs=pl.BlockSpec((1, _B // 128, 128), lambda i: (i, 0, 0)),
        out_shape=jax.ShapeDtypeStruct((_NB, _B // 128, 128), jnp.int32),
    )(xt, g2, norm2)


def _sc_gather(idx_flat, table_t):
    mesh = plsc.VectorSubcoreMesh(core_axis_name="c", subcore_axis_name="s")

    @functools.partial(
        pl.kernel,
        mesh=mesh,
        out_type=jax.ShapeDtypeStruct((_K, _N), jnp.float32),
        scratch_types=[
            pltpu.VMEM((_ROWS_PER_W,), jnp.int32),
            [pltpu.VMEM((_C,), jnp.float32) for _ in range(_K)],
            pltpu.VMEM((_K, _CHUNK), jnp.float32),
        ],
        compiler_params=pltpu.CompilerParams(needs_layout_passes=False),
    )
    def k(idx_hbm, table_hbm, out_hbm, idx_v, tabs, rows_v):
        wid = lax.axis_index("s") * _NC + lax.axis_index("c")
        row_base = wid * _ROWS_PER_W
        for kk in range(_K):
            pltpu.sync_copy(table_hbm.at[kk], tabs[kk])
        pltpu.sync_copy(
            idx_hbm.at[pl.ds(pl.multiple_of(row_base, 128), _ROWS_PER_W)],
            idx_v)

        def chunk(c, carry):
            def body16(c2, carry2):
                coff = pl.multiple_of(c * _CHUNK + c2 * 16, 8)
                rvec = idx_v[pl.ds(coff, 16)]
                soff = pl.multiple_of(c2 * 16, 8)
                for kk in range(_K):
                    rows_v[kk, pl.ds(soff, 16)] = plsc.load_gather(
                        tabs[kk], [rvec])
                return carry2

            lax.fori_loop(0, _CHUNK // 16, body16, 0, unroll=4)
            ooff = pl.multiple_of(row_base + c * _CHUNK, 128)
            pltpu.sync_copy(rows_v, out_hbm.at[:, pl.ds(ooff, _CHUNK)])
            return carry

        lax.fori_loop(0, _ROWS_PER_W // _CHUNK, chunk, 0)

    return k(idx_flat, table_t)


def kernel(X, grid, grid_norm):
    xt = X.T
    g2 = 2.0 * grid
    norm2 = grid_norm.reshape(_C, 1)
    idx3 = _tc_score_argmax(xt, g2, norm2)
    idx_flat = idx3.reshape(_N)
    quantized = _sc_gather(idx_flat, grid.T)
    return (quantized.T, idx_flat.astype(jnp.uint8))


# speedup vs baseline: 2.9032x; 1.3580x over previous
import functools

import jax
import jax.numpy as jnp
from jax import lax
from jax.experimental import pallas as pl
from jax.experimental.pallas import tpu as pltpu
from jax.experimental.pallas import tpu_sc as plsc

_N = 524288
_K = 8
_C = 256
_B = 8192
_NB = _N // _B

_NC = 2
_NS = 16
_NW = _NC * _NS
_ROWS_PER_W = _N // _NW
_CHUNK = 4096


def _score_argmax_body(xt_ref, g2_ref, norm_ref, idx_ref):
    sc = jnp.dot(g2_ref[...], xt_ref[...], preferred_element_type=jnp.float32)
    sc = sc - norm_ref[...]
    m = lax.slice(sc, (0, 0), (8, _B))
    tof = jnp.zeros((8, _B), jnp.int32)
    for t in range(1, _C // 8):
        v = lax.slice(sc, (t * 8, 0), (t * 8 + 8, _B))
        cond = v > m
        m = jnp.where(cond, v, m)
        tof = jnp.where(cond, t, tof)
    fm = jnp.max(m, axis=0, keepdims=True)
    j = tof * 8 + lax.broadcasted_iota(jnp.int32, (8, _B), 0)
    cand = jnp.where(m == fm, j, _C)
    idx_ref[0] = jnp.min(cand, axis=0).reshape(_B // 128, 128)


def _tc_score_argmax(xt, g2, norm2):
    return pl.pallas_call(
        _score_argmax_body,
        grid=(_NB,),
        in_specs=[
            pl.BlockSpec((_K, _B), lambda i: (0, i)),
            pl.BlockSpec((_C, _K), lambda i: (0, 0)),
            pl.BlockSpec((_C, 1), lambda i: (0, 0)),
        ],
        out_specs=pl.BlockSpec((1, _B // 128, 128), lambda i: (i, 0, 0)),
        out_shape=jax.ShapeDtypeStruct((_NB, _B // 128, 128), jnp.int32),
    )(xt, g2, norm2)


def _sc_gather(idx_flat, table_t):
    mesh = plsc.VectorSubcoreMesh(core_axis_name="c", subcore_axis_name="s")

    @functools.partial(
        pl.kernel,
        mesh=mesh,
        out_type=jax.ShapeDtypeStruct((_K, _N), jnp.float32),
        scratch_types=[
            pltpu.VMEM((_ROWS_PER_W,), jnp.int32),
            [pltpu.VMEM((_C,), jnp.float32) for _ in range(_K)],
            pltpu.VMEM((_K, _CHUNK), jnp.float32),
        ],
        compiler_params=pltpu.CompilerParams(needs_layout_passes=False),
    )
    def k(idx_hbm, table_hbm, out_hbm, idx_v, tabs, rows_v):
        wid = lax.axis_index("s") * _NC + lax.axis_index("c")
        row_base = wid * _ROWS_PER_W
        for kk in range(_K):
            pltpu.sync_copy(table_hbm.at[kk], tabs[kk])
        pltpu.sync_copy(
            idx_hbm.at[pl.ds(pl.multiple_of(row_base, 128), _ROWS_PER_W)],
            idx_v)

        def chunk(c, carry):
            def body16(c2, carry2):
                coff = pl.multiple_of(c * _CHUNK + c2 * 16, 8)
                rvec = idx_v[pl.ds(coff, 16)]
                soff = pl.multiple_of(c2 * 16, 8)
                for kk in range(_K):
                    rows_v[kk, pl.ds(soff, 16)] = plsc.load_gather(
                        tabs[kk], [rvec])
                return carry2

            lax.fori_loop(0, _CHUNK // 16, body16, 0, unroll=4)
            ooff = pl.multiple_of(row_base + c * _CHUNK, 128)
            pltpu.sync_copy(rows_v, out_hbm.at[:, pl.ds(ooff, _CHUNK)])
            return carry

        lax.fori_loop(0, _ROWS_PER_W // _CHUNK, chunk, 0)

    return k(idx_flat, table_t)


def kernel(X, grid, grid_norm):
    xt = X.T
    g2 = 2.0 * grid
    norm2 = grid_norm.reshape(_C, 1)
    idx3 = _tc_score_argmax(xt, g2, norm2)
    idx_flat = idx3.reshape(_N)
    quantized = _sc_gather(idx_flat, grid.T)
    return (quantized.T, idx_flat.astype(jnp.uint8))


# speedup vs baseline: 3.1512x; 1.0854x over previous
import functools

import jax
import jax.numpy as jnp
from jax import lax
from jax.experimental import pallas as pl
from jax.experimental.pallas import tpu as pltpu
from jax.experimental.pallas import tpu_sc as plsc

_N = 524288
_K = 8
_C = 256
_B = 8192
_NSEG = 4
_NSEG_ROWS = _N // _NSEG
_NB_SEG = _NSEG_ROWS // _B

_NC = 2
_NS = 16
_NW = _NC * _NS
_ROWS_PER_W = _NSEG_ROWS // _NW
_CHUNK = min(4096, _ROWS_PER_W)


def _score_argmax_body(xt_ref, g2_ref, norm_ref, idx_ref):
    sc = jnp.dot(g2_ref[...], xt_ref[...], preferred_element_type=jnp.float32)
    sc = sc - norm_ref[...]
    m = lax.slice(sc, (0, 0), (8, _B))
    tof = jnp.zeros((8, _B), jnp.int32)
    for t in range(1, _C // 8):
        v = lax.slice(sc, (t * 8, 0), (t * 8 + 8, _B))
        cond = v > m
        m = jnp.where(cond, v, m)
        tof = jnp.where(cond, t, tof)
    fm = jnp.max(m, axis=0, keepdims=True)
    j = tof * 8 + lax.broadcasted_iota(jnp.int32, (8, _B), 0)
    cand = jnp.where(m == fm, j, _C)
    idx_ref[0] = jnp.min(cand, axis=0).reshape(_B // 128, 128)


def _tc_score_argmax(xt, g2, norm2, seg):
    base = seg * _NB_SEG
    return pl.pallas_call(
        _score_argmax_body,
        grid=(_NB_SEG,),
        in_specs=[
            pl.BlockSpec((_K, _B), lambda i: (0, base + i)),
            pl.BlockSpec((_C, _K), lambda i: (0, 0)),
            pl.BlockSpec((_C, 1), lambda i: (0, 0)),
        ],
        out_specs=pl.BlockSpec((1, _B // 128, 128), lambda i: (i, 0, 0)),
        out_shape=jax.ShapeDtypeStruct((_NB_SEG, _B // 128, 128), jnp.int32),
    )(xt, g2, norm2)


def _sc_gather(idx_flat, table_t):
    mesh = plsc.VectorSubcoreMesh(core_axis_name="c", subcore_axis_name="s")

    @functools.partial(
        pl.kernel,
        mesh=mesh,
        out_type=jax.ShapeDtypeStruct((_K, _NSEG_ROWS), jnp.float32),
        scratch_types=[
            pltpu.VMEM((_ROWS_PER_W,), jnp.int32),
            [pltpu.VMEM((_C,), jnp.float32) for _ in range(_K)],
            pltpu.VMEM((_K, _CHUNK), jnp.float32),
        ],
        compiler_params=pltpu.CompilerParams(needs_layout_passes=False),
    )
    def k(idx_hbm, table_hbm, out_hbm, idx_v, tabs, rows_v):
        wid = lax.axis_index("s") * _NC + lax.axis_index("c")
        row_base = wid * _ROWS_PER_W
        for kk in range(_K):
            pltpu.sync_copy(table_hbm.at[kk], tabs[kk])
        pltpu.sync_copy(
            idx_hbm.at[pl.ds(pl.multiple_of(row_base, 128), _ROWS_PER_W)],
            idx_v)

        def chunk(c, carry):
            def body16(c2, carry2):
                coff = pl.multiple_of(c * _CHUNK + c2 * 16, 8)
                rvec = idx_v[pl.ds(coff, 16)]
                soff = pl.multiple_of(c2 * 16, 8)
                for kk in range(_K):
                    rows_v[kk, pl.ds(soff, 16)] = plsc.load_gather(
                        tabs[kk], [rvec])
                return carry2

            lax.fori_loop(0, _CHUNK // 16, body16, 0, unroll=4)
            ooff = pl.multiple_of(row_base + c * _CHUNK, 128)
            pltpu.sync_copy(rows_v, out_hbm.at[:, pl.ds(ooff, _CHUNK)])
            return carry

        lax.fori_loop(0, _ROWS_PER_W // _CHUNK, chunk, 0)

    return k(idx_flat, table_t)


def kernel(X, grid, grid_norm):
    xt = X.T
    g2 = 2.0 * grid
    norm2 = grid_norm.reshape(_C, 1)
    gt = grid.T
    quant_segs, idx_segs = [], []
    for seg in range(_NSEG):
        idx3 = _tc_score_argmax(xt, g2, norm2, seg)
        idx_flat = idx3.reshape(_NSEG_ROWS)
        quant_segs.append(_sc_gather(idx_flat, gt))
        idx_segs.append(idx_flat.astype(jnp.uint8))
    quantized = jnp.concatenate(quant_segs, axis=1)
    return (quantized.T, jnp.concatenate(idx_segs))


# speedup vs baseline: 3.1842x; 1.0105x over previous
import functools

import jax
import jax.numpy as jnp
from jax import lax
from jax.experimental import pallas as pl
from jax.experimental.pallas import tpu as pltpu
from jax.experimental.pallas import tpu_sc as plsc

_N = 524288
_K = 8
_C = 256
_B = 16384
_NSEG = 4
_NSEG_ROWS = _N // _NSEG
_NB_SEG = _NSEG_ROWS // _B

_NC = 2
_NS = 16
_NW = _NC * _NS
_ROWS_PER_W = _NSEG_ROWS // _NW
_CHUNK = min(4096, _ROWS_PER_W)


def _score_argmax_body(xt_ref, g2_ref, norm_ref, idx_ref):
    sc = jnp.dot(g2_ref[...], xt_ref[...], preferred_element_type=jnp.float32)
    sc = sc - norm_ref[...]
    m = lax.slice(sc, (0, 0), (8, _B))
    tof = jnp.zeros((8, _B), jnp.int32)
    for t in range(1, _C // 8):
        v = lax.slice(sc, (t * 8, 0), (t * 8 + 8, _B))
        cond = v > m
        m = jnp.where(cond, v, m)
        tof = jnp.where(cond, t, tof)
    fm = jnp.max(m, axis=0, keepdims=True)
    j = tof * 8 + lax.broadcasted_iota(jnp.int32, (8, _B), 0)
    cand = jnp.where(m == fm, j, _C)
    idx_ref[0] = jnp.min(cand, axis=0).reshape(_B // 128, 128)


def _tc_score_argmax(xt, g2, norm2, seg):
    base = seg * _NB_SEG
    return pl.pallas_call(
        _score_argmax_body,
        grid=(_NB_SEG,),
        in_specs=[
            pl.BlockSpec((_K, _B), lambda i: (0, base + i)),
            pl.BlockSpec((_C, _K), lambda i: (0, 0)),
            pl.BlockSpec((_C, 1), lambda i: (0, 0)),
        ],
        out_specs=pl.BlockSpec((1, _B // 128, 128), lambda i: (i, 0, 0)),
        out_shape=jax.ShapeDtypeStruct((_NB_SEG, _B // 128, 128), jnp.int32),
    )(xt, g2, norm2)


def _sc_gather(idx_flat, table_t):
    mesh = plsc.VectorSubcoreMesh(core_axis_name="c", subcore_axis_name="s")

    @functools.partial(
        pl.kernel,
        mesh=mesh,
        out_type=jax.ShapeDtypeStruct((_K, _NSEG_ROWS), jnp.float32),
        scratch_types=[
            pltpu.VMEM((_ROWS_PER_W,), jnp.int32),
            [pltpu.VMEM((_C,), jnp.float32) for _ in range(_K)],
            pltpu.VMEM((_K, _CHUNK), jnp.float32),
        ],
        compiler_params=pltpu.CompilerParams(needs_layout_passes=False),
    )
    def k(idx_hbm, table_hbm, out_hbm, idx_v, tabs, rows_v):
        wid = lax.axis_index("s") * _NC + lax.axis_index("c")
        row_base = wid * _ROWS_PER_W
        for kk in range(_K):
            pltpu.sync_copy(table_hbm.at[kk], tabs[kk])
        pltpu.sync_copy(
            idx_hbm.at[pl.ds(pl.multiple_of(row_base, 128), _ROWS_PER_W)],
            idx_v)

        def chunk(c, carry):
            def body16(c2, carry2):
                coff = pl.multiple_of(c * _CHUNK + c2 * 16, 8)
                rvec = idx_v[pl.ds(coff, 16)]
                soff = pl.multiple_of(c2 * 16, 8)
                for kk in range(_K):
                    rows_v[kk, pl.ds(soff, 16)] = plsc.load_gather(
                        tabs[kk], [rvec])
                return carry2

            lax.fori_loop(0, _CHUNK // 16, body16, 0, unroll=8)
            ooff = pl.multiple_of(row_base + c * _CHUNK, 128)
            pltpu.sync_copy(rows_v, out_hbm.at[:, pl.ds(ooff, _CHUNK)])
            return carry

        lax.fori_loop(0, _ROWS_PER_W // _CHUNK, chunk, 0)

    return k(idx_flat, table_t)


def kernel(X, grid, grid_norm):
    xt = X.T
    g2 = 2.0 * grid
    norm2 = grid_norm.reshape(_C, 1)
    gt = grid.T
    quant_segs, idx_segs = [], []
    for seg in range(_NSEG):
        idx3 = _tc_score_argmax(xt, g2, norm2, seg)
        idx_flat = idx3.reshape(_NSEG_ROWS)
        quant_segs.append(_sc_gather(idx_flat, gt))
        idx_segs.append(idx_flat.astype(jnp.uint8))
    quantized = jnp.concatenate(quant_segs, axis=1)
    return (quantized.T, jnp.concatenate(idx_segs))


# speedup vs baseline: 3.3205x; 1.0428x over previous
import functools

import jax
import jax.numpy as jnp
from jax import lax
from jax.experimental import pallas as pl
from jax.experimental.pallas import tpu as pltpu
from jax.experimental.pallas import tpu_sc as plsc

_N = 524288
_K = 8
_C = 256
_B = 16384
_NSEG = 4
_NSEG_ROWS = _N // _NSEG
_NB_SEG = _NSEG_ROWS // _B

_NC = 2
_NS = 16
_NW = _NC * _NS
_ROWS_PER_W = _NSEG_ROWS // _NW
_CHUNK = min(4096, _ROWS_PER_W)


def _score_argmax_body(xt_ref, g2_ref, norm_ref, idx_ref):
    sc = jnp.dot(g2_ref[...], xt_ref[...], preferred_element_type=jnp.float32)
    sc = sc - norm_ref[...]
    m = lax.slice(sc, (0, 0), (8, _B))
    tof = jnp.zeros((8, _B), jnp.int32)
    for t in range(1, _C // 8):
        v = lax.slice(sc, (t * 8, 0), (t * 8 + 8, _B))
        cond = v > m
        m = jnp.where(cond, v, m)
        tof = jnp.where(cond, t, tof)
    fm = jnp.max(m, axis=0, keepdims=True)
    j = tof * 8 + lax.broadcasted_iota(jnp.int32, (8, _B), 0)
    cand = jnp.where(m == fm, j, _C)
    idx_ref[0] = jnp.min(cand, axis=0).reshape(_B // 128, 128)


def _tc_score_argmax(xt, g2, norm2, seg):
    base = seg * _NB_SEG
    return pl.pallas_call(
        _score_argmax_body,
        grid=(_NB_SEG,),
        in_specs=[
            pl.BlockSpec((_K, _B), lambda i: (0, base + i)),
            pl.BlockSpec((_C, _K), lambda i: (0, 0)),
            pl.BlockSpec((_C, 1), lambda i: (0, 0)),
        ],
        out_specs=pl.BlockSpec((1, _B // 128, 128), lambda i: (i, 0, 0)),
        out_shape=jax.ShapeDtypeStruct((_NB_SEG, _B // 128, 128), jnp.int32),
    )(xt, g2, norm2)


def _sc_gather(idx_flat, table_t, out_cols=_NSEG_ROWS):
    mesh = plsc.VectorSubcoreMesh(core_axis_name="c", subcore_axis_name="s")

    @functools.partial(
        pl.kernel,
        mesh=mesh,
        out_type=jax.ShapeDtypeStruct((_K, out_cols), jnp.float32),
        scratch_types=[
            pltpu.VMEM((_ROWS_PER_W,), jnp.int32),
            [pltpu.VMEM((_C,), jnp.float32) for _ in range(_K)],
            pltpu.VMEM((_K, _CHUNK), jnp.float32),
        ],
        compiler_params=pltpu.CompilerParams(needs_layout_passes=False),
    )
    def k(idx_hbm, table_hbm, out_hbm, idx_v, tabs, rows_v):
        wid = lax.axis_index("s") * _NC + lax.axis_index("c")
        row_base = wid * _ROWS_PER_W
        for kk in range(_K):
            pltpu.sync_copy(table_hbm.at[kk], tabs[kk])
        pltpu.sync_copy(
            idx_hbm.at[pl.ds(pl.multiple_of(row_base, 128), _ROWS_PER_W)],
            idx_v)

        def chunk(c, carry):
            def body16(c2, carry2):
                coff = pl.multiple_of(c * _CHUNK + c2 * 16, 8)
                rvec = idx_v[pl.ds(coff, 16)]
                soff = pl.multiple_of(c2 * 16, 8)
                for kk in range(_K):
                    rows_v[kk, pl.ds(soff, 16)] = plsc.load_gather(
                        tabs[kk], [rvec])
                return carry2

            lax.fori_loop(0, _CHUNK // 16, body16, 0, unroll=8)
            ooff = pl.multiple_of(row_base + c * _CHUNK, 128)
            pltpu.sync_copy(rows_v, out_hbm.at[:, pl.ds(ooff, _CHUNK)])
            return carry

        lax.fori_loop(0, _ROWS_PER_W // _CHUNK, chunk, 0)

    return k(idx_flat, table_t)


def kernel(X, grid, grid_norm):
    xt = X.T
    g2 = 2.0 * grid
    norm2 = grid_norm.reshape(_C, 1)
    gt = grid.T
    quantized = None
    idx_segs = []
    for seg in range(_NSEG):
        idx3 = _tc_score_argmax(xt, g2, norm2, seg)
        idx_flat = idx3.reshape(_NSEG_ROWS)
        if seg == 0:
            quantized = _sc_gather(idx_flat, gt, out_cols=_N)
        else:
            qs = _sc_gather(idx_flat, gt)
            quantized = lax.dynamic_update_slice(
                quantized, qs, (0, seg * _NSEG_ROWS))
        idx_segs.append(idx_flat.astype(jnp.uint8))
    return (quantized.T, jnp.concatenate(idx_segs))


# speedup vs baseline: 3.5495x; 1.0690x over previous
import functools

import jax
import jax.numpy as jnp
from jax import lax
from jax.experimental import pallas as pl
from jax.experimental.pallas import tpu as pltpu
from jax.experimental.pallas import tpu_sc as plsc

_N = 524288
_K = 8
_C = 256
_B = 16384
_NB = _N // _B
_SEG_BLOCKS = (10, 10, 10, 2)

_NC = 2
_NS = 16
_NW = _NC * _NS


def _score_argmax_body(xt_ref, aug_ref, idx_ref):
    aug = aug_ref[...]
    g2 = lax.slice(aug, (0, 0), (_C, _K))
    norm = lax.slice(aug, (0, _K), (_C, _K + 1))
    sc = jnp.dot(g2, xt_ref[...], preferred_element_type=jnp.float32)
    sc = sc - norm
    m = lax.slice(sc, (0, 0), (8, _B))
    tof = jnp.zeros((8, _B), jnp.int32)
    for t in range(1, _C // 8):
        v = lax.slice(sc, (t * 8, 0), (t * 8 + 8, _B))
        cond = v > m
        m = jnp.where(cond, v, m)
        tof = jnp.where(cond, t, tof)
    fm = jnp.max(m, axis=0, keepdims=True)
    j = tof * 8 + lax.broadcasted_iota(jnp.int32, (8, _B), 0)
    cand = jnp.where(m == fm, j, _C)
    idx_ref[0] = jnp.min(cand, axis=0).reshape(_B // 128, 128)


def _tc_score_argmax(xt, aug, base, nblocks):
    return pl.pallas_call(
        _score_argmax_body,
        grid=(nblocks,),
        in_specs=[
            pl.BlockSpec((_K, _B), lambda i: (0, base + i)),
            pl.BlockSpec((_C, 16), lambda i: (0, 0)),
        ],
        out_specs=pl.BlockSpec((1, _B // 128, 128), lambda i: (i, 0, 0)),
        out_shape=jax.ShapeDtypeStruct((nblocks, _B // 128, 128), jnp.int32),
    )(xt, aug)


def _sc_gather(idx_flat, table_t, out_cols):
    n = idx_flat.shape[0]
    rows_per_w = n // _NW
    chunk = min(4096, rows_per_w)
    mesh = plsc.VectorSubcoreMesh(core_axis_name="c", subcore_axis_name="s")

    @functools.partial(
        pl.kernel,
        mesh=mesh,
        out_type=jax.ShapeDtypeStruct((_K, out_cols), jnp.float32),
        scratch_types=[
            pltpu.VMEM((rows_per_w,), jnp.int32),
            [pltpu.VMEM((_C,), jnp.float32) for _ in range(_K)],
            pltpu.VMEM((_K, chunk), jnp.float32),
            pltpu.SemaphoreType.DMA,
        ],
        compiler_params=pltpu.CompilerParams(needs_layout_passes=False),
    )
    def k(idx_hbm, table_hbm, out_hbm, idx_v, tabs, rows_v, sem):
        wid = lax.axis_index("s") * _NC + lax.axis_index("c")
        row_base = wid * rows_per_w
        copies = [pltpu.async_copy(table_hbm.at[kk], tabs[kk], sem)
                  for kk in range(_K)]
        copies.append(pltpu.async_copy(
            idx_hbm.at[pl.ds(pl.multiple_of(row_base, 128), rows_per_w)],
            idx_v, sem))
        for cp in copies:
            cp.wait()

        def do_chunk(c, carry):
            def body16(c2, carry2):
                coff = pl.multiple_of(c * chunk + c2 * 16, 8)
                rvec = idx_v[pl.ds(coff, 16)]
                soff = pl.multiple_of(c2 * 16, 8)
                for kk in range(_K):
                    rows_v[kk, pl.ds(soff, 16)] = plsc.load_gather(
                        tabs[kk], [rvec])
                return carry2

            lax.fori_loop(0, chunk // 16, body16, 0, unroll=8)
            ooff = pl.multiple_of(row_base + c * chunk, 128)
            pltpu.sync_copy(rows_v, out_hbm.at[:, pl.ds(ooff, chunk)])
            return carry

        lax.fori_loop(0, rows_per_w // chunk, do_chunk, 0)

    return k(idx_flat, table_t)


def kernel(X, grid, grid_norm):
    xt = X.T
    gt = grid.T
    aug = jnp.concatenate(
        [2.0 * grid, grid_norm.reshape(_C, 1),
         jnp.zeros((_C, 16 - _K - 1), jnp.float32)], axis=1)
    quantized = None
    idx_segs = []
    base = 0
    for seg, nblocks in enumerate(_SEG_BLOCKS):
        seg_rows = nblocks * _B
        idx3 = _tc_score_argmax(xt, aug, base, nblocks)
        idx_flat = idx3.reshape(seg_rows)
        if seg == 0:
            quantized = _sc_gather(idx_flat, gt, _N)
        else:
            qs = _sc_gather(idx_flat, gt, seg_rows)
            quantized = lax.dynamic_update_slice(
                quantized, qs, (0, base * _B))
        idx_segs.append(idx_flat.astype(jnp.uint8))
        base += nblocks
    return (quantized.T, jnp.concatenate(idx_segs))
